# baseline (speedup 1.0000x reference)
import functools

import jax
import jax.numpy as jnp
from jax import lax
from jax.experimental import pallas as pl
from jax.experimental.pallas import tpu as pltpu
from jax.experimental.pallas import tpu_sc as plsc

SEQ = 1024
SLEN = 50
SPAD = 56
DIM = 768
NW = 32
SEQ_PER_W = SEQ // NW

_mesh = plsc.VectorSubcoreMesh(core_axis_name="c", subcore_axis_name="s")


@functools.partial(
    pl.kernel,
    mesh=_mesh,
    out_type=jax.ShapeDtypeStruct((SEQ, SPAD, DIM), jnp.float32),
    scratch_types=[
        pltpu.VMEM((SEQ_PER_W * SPAD,), jnp.int32),
        pltpu.VMEM((SPAD, DIM), jnp.float32),
        pltpu.VMEM((SPAD, DIM), jnp.float32),
        pltpu.SemaphoreType.DMA,
        pltpu.SemaphoreType.DMA,
        pltpu.SemaphoreType.DMA,
        pltpu.SemaphoreType.DMA,
    ],
)
def _gather(table_hbm, idx_hbm, out_hbm, idx_v, rows0, rows1,
            gsem0, gsem1, osem0, osem1):
    wid = lax.axis_index("s") * 2 + lax.axis_index("c")
    base = wid * SEQ_PER_W
    pltpu.sync_copy(idx_hbm.at[pl.ds(base * SPAD, SEQ_PER_W * SPAD)], idx_v)

    bufs = (rows0, rows1)
    gsems = (gsem0, gsem1)
    osems = (osem0, osem1)

    def idx_slice(s):
        return idx_v.at[pl.ds(pl.multiple_of(s * SPAD, 8), SPAD)]

    def g_start(s, j):
        pltpu.async_copy(table_hbm.at[idx_slice(s)], bufs[j], gsems[j])

    def g_wait(s, j):
        pltpu.make_async_copy(table_hbm.at[idx_slice(s)], bufs[j], gsems[j]).wait()

    def ow_start(s, j):
        pltpu.async_copy(bufs[j], out_hbm.at[base + s], osems[j])

    def ow_wait(s, j):
        pltpu.make_async_copy(bufs[j], out_hbm.at[base + s], osems[j]).wait()

    g_start(0, 0)
    g_start(1, 1)

    def body(c2, _):
        s = c2 * 2
        for j in range(2):
            g_wait(s + j, j)
            ow_start(s + j, j)
        for j in range(2):
            ow_wait(s + j, j)
            g_start(s + 2 + j, j)
        return 0

    lax.fori_loop(0, SEQ_PER_W // 2 - 1, body, 0)

    s = SEQ_PER_W - 2
    for j in range(2):
        g_wait(s + j, j)
        ow_start(s + j, j)
    for j in range(2):
        ow_wait(s + j, j)


def kernel(x, embedding_weight):
    idx = jnp.pad(x, ((0, 0), (0, SPAD - SLEN))).reshape(-1)
    out = _gather(embedding_weight, idx)
    return out[:, :SLEN, :]
